# baseline (speedup 1.0000x reference)
import functools

import jax
import jax.numpy as jnp
from jax import lax
from jax.experimental import pallas as pl
from jax.experimental.pallas import tpu as pltpu
from jax.experimental.pallas import tpu_sc as plsc

_N = 16384
_D = 128
_NC = 2
_NS = 16
_NW = _NC * _NS
_ROWS_PER_W = _N // _NW
_C = 64
_NCHUNK = _ROWS_PER_W // _C
_NIN = 3
_NOUT = 2
_LANE = 16
_VECS_PER_ROW = _D // _LANE


def _mix_body(x_hbm, perm_hbm, alpha_hbm, out_hbm, idx_v, x_v, p_v, o_v,
              alpha_s, sem_a, sem_i, sem_i0, sem_i1, sem_i2, sem_o0, sem_o1):
    wid = lax.axis_index("s") * _NC + lax.axis_index("c")
    base_w = wid * _ROWS_PER_W
    si = (sem_i0, sem_i1, sem_i2)
    so = (sem_o0, sem_o1)

    c_a = pltpu.async_copy(alpha_hbm, alpha_s, sem_a)
    c_i = pltpu.async_copy(perm_hbm.at[wid], idx_v, sem_i)

    def lin_in(ci):
        b = ci % _NIN
        return pltpu.async_copy(x_hbm.at[pl.ds(base_w + ci * _C, _C)],
                                x_v.at[b], si[b])

    def gat_in(ci):
        b = ci % _NIN
        return pltpu.async_copy(x_hbm.at[idx_v.at[ci]], p_v.at[b], si[b])

    cxs = [lin_in(ci) for ci in range(_NIN)]
    c_i.wait()
    ins = [None] * _NCHUNK
    for ci in range(_NIN):
        ins[ci] = (gat_in(ci), cxs[ci])

    def issue_in(ci):
        return gat_in(ci), lin_in(ci)

    c_a.wait()
    av = alpha_s[...]
    bv = jnp.float32(1.0) - av

    def compute(bi, bo):
        def row(r, c2):
            for j in range(_VECS_PER_ROW):
                sl = pl.ds(j * _LANE, _LANE)
                o_v[bo, r, sl] = av * x_v[bi, r, sl] + bv * p_v[bi, r, sl]
            return c2
        lax.fori_loop(0, _C, row, 0)

    outs = [None] * _NCHUNK
    for ci in range(_NCHUNK):
        bi = ci % _NIN
        bo = ci % _NOUT
        cg, cx = ins[ci]
        cg.wait()
        cx.wait()
        if ci >= _NOUT:
            outs[ci - _NOUT].wait()
        compute(bi, bo)
        outs[ci] = pltpu.async_copy(
            o_v.at[bo], out_hbm.at[pl.ds(base_w + ci * _C, _C)], so[bo])
        if ci + _NIN < _NCHUNK:
            ins[ci + _NIN] = issue_in(ci + _NIN)
    outs[_NCHUNK - 2].wait()
    outs[_NCHUNK - 1].wait()


@functools.partial(jax.jit)
def _mix(x, perm3, alpha1):
    mesh = plsc.VectorSubcoreMesh(core_axis_name="c", subcore_axis_name="s")
    return pl.kernel(
        _mix_body,
        mesh=mesh,
        out_type=jax.ShapeDtypeStruct((_N, _D), jnp.float32),
        scratch_types=[
            pltpu.VMEM((_NCHUNK, _C), jnp.int32),
            pltpu.VMEM((_NIN, _C, _D), jnp.float32),
            pltpu.VMEM((_NIN, _C, _D), jnp.float32),
            pltpu.VMEM((_NOUT, _C, _D), jnp.float32),
            pltpu.VMEM((_LANE,), jnp.float32),
            pltpu.SemaphoreType.DMA,
            pltpu.SemaphoreType.DMA,
            pltpu.SemaphoreType.DMA,
            pltpu.SemaphoreType.DMA,
            pltpu.SemaphoreType.DMA,
            pltpu.SemaphoreType.DMA,
            pltpu.SemaphoreType.DMA,
        ],
    )(x, perm3, alpha1)


def kernel(x, perm, alpha):
    perm3 = perm.astype(jnp.int32).reshape(_NW, _NCHUNK, _C)
    alpha1 = jnp.full((_LANE,), alpha, jnp.float32)
    return _mix(x, perm3, alpha1)
